# baseline (speedup 1.0000x reference)
import jax
import jax.numpy as jnp
from jax.experimental import pallas as pl
from jax.experimental.pallas import tpu as pltpu

ALPHA = 0.05
ROWS = 64
FEAT = 4096
N = 64
TILE = 512
NTILES = FEAT // TILE
BATCH = 8


def _body(xr_ref, a_ref, w0_ref, w1_ref, b0_ref, b1_ref, wm_ref, bm_ref,
          out_ref, h1_ref, h1b_ref, t1_ref, s_ref, sx_ref):
    j = pl.program_id(0)

    @pl.when(j == 0)
    def _compute_s():
        a = a_ref[...]
        mask = (a != 0.0).astype(jnp.float32)
        ii = jax.lax.broadcasted_iota(jnp.int32, (N, N), 0)
        jj = jax.lax.broadcasted_iota(jnp.int32, (N, N), 1)
        eye = (ii == jj).astype(jnp.float32)
        pad = jnp.float32(N * N) - jnp.sum(mask)
        delta00 = ((ii == 0) & (jj == 0)).astype(jnp.float32)
        cnt = mask + eye + pad * delta00
        deg = jnp.sum(cnt, axis=0, keepdims=True)
        dinv = jax.lax.rsqrt(deg)
        s = cnt.T * dinv * dinv.reshape(N, 1)
        s_ref[...] = s
        sx_ref[...] = jnp.dot(s, xr_ref[...],
                              preferred_element_type=jnp.float32).astype(jnp.bfloat16)

    @pl.when(j < NTILES)
    def _layer0():
        dsj = pl.ds(j * TILE, TILE)
        t = jnp.dot(sx_ref[...], w0_ref[...].astype(jnp.bfloat16),
                    preferred_element_type=jnp.float32)
        h1_tile = ALPHA * xr_ref[:, dsj] + t + b0_ref[:, dsj]
        h1_ref[:, dsj] = h1_tile
        h1b_ref[:, dsj] = h1_tile.astype(jnp.bfloat16)

    @pl.when(j > 0)
    def _layer1():
        dsp = pl.ds((j - 1) * TILE, TILE)
        part = jnp.dot(h1b_ref[:, dsp], w1_ref[...].astype(jnp.bfloat16),
                       preferred_element_type=jnp.float32)

        @pl.when(j == 1)
        def _init():
            t1_ref[...] = part

        @pl.when(j > 1)
        def _acc():
            t1_ref[...] = t1_ref[...] + part

    @pl.when(j == NTILES)
    def _epilogue():
        h2 = (ALPHA * xr_ref[...]
              + jnp.dot(s_ref[...], t1_ref[...], preferred_element_type=jnp.float32)
              + b1_ref[...])
        wm = wm_ref[...]
        bm = bm_ref[...].reshape(64, 1)
        g0 = xr_ref[...].reshape(BATCH, 64, 512)
        g1 = h1_ref[...].reshape(BATCH, 64, 512)
        g2 = h2.reshape(BATCH, 64, 512)
        for b in range(BATCH):
            ob = (jnp.dot(wm[:, 0:64], g0[b], preferred_element_type=jnp.float32)
                  + jnp.dot(wm[:, 64:128], g1[b], preferred_element_type=jnp.float32)
                  + jnp.dot(wm[:, 128:192], g2[b], preferred_element_type=jnp.float32)
                  + bm)
            out_ref[pl.ds(b * 8, 8), :] = ob.reshape(8, FEAT)


def kernel(X, A, W_g0, b_g0, W_g1, b_g1, W_mlp, b_mlp):
    batch, c, n, seq = X.shape
    Xr = X.reshape(ROWS, FEAT)
    out_r = pl.pallas_call(
        _body,
        grid=(NTILES + 1,),
        in_specs=[
            pl.BlockSpec((ROWS, FEAT), lambda j: (0, 0)),
            pl.BlockSpec((N, N), lambda j: (0, 0)),
            pl.BlockSpec((FEAT, TILE),
                         lambda j: (0, jnp.minimum(j, NTILES - 1))),
            pl.BlockSpec((TILE, FEAT),
                         lambda j: (jnp.maximum(j - 1, 0), 0)),
            pl.BlockSpec((1, FEAT), lambda j: (0, 0)),
            pl.BlockSpec((1, FEAT), lambda j: (0, 0)),
            pl.BlockSpec((64, 192), lambda j: (0, 0)),
            pl.BlockSpec((1, 64), lambda j: (0, 0)),
        ],
        out_specs=pl.BlockSpec((ROWS, FEAT), lambda j: (0, 0)),
        out_shape=jax.ShapeDtypeStruct((ROWS, FEAT), jnp.float32),
        scratch_shapes=[
            pltpu.VMEM((ROWS, FEAT), jnp.float32),
            pltpu.VMEM((ROWS, FEAT), jnp.bfloat16),
            pltpu.VMEM((ROWS, FEAT), jnp.float32),
            pltpu.VMEM((N, N), jnp.float32),
            pltpu.VMEM((ROWS, FEAT), jnp.bfloat16),
        ],
    )(Xr, A, W_g0, W_g1, b_g0.reshape(1, FEAT), b_g1.reshape(1, FEAT),
      W_mlp, b_mlp.reshape(1, 64))
    return out_r.reshape(batch, c, n, seq)


# speedup vs baseline: 1.0252x; 1.0252x over previous
import jax
import jax.numpy as jnp
from jax.experimental import pallas as pl
from jax.experimental.pallas import tpu as pltpu

ALPHA = 0.05
ROWS = 64
FEAT = 4096
N = 64
TILE = 256
NTILES = FEAT // TILE
BATCH = 8


def _body(xr_ref, a_ref, w0_ref, w1_ref, b0_ref, b1_ref, wm_ref, bm_ref,
          out_ref, h1_ref, h1b_ref, t1_ref, s_ref, sx_ref):
    j = pl.program_id(0)

    @pl.when(j == 0)
    def _compute_s():
        a = a_ref[...]
        mask = (a != 0.0).astype(jnp.float32)
        ii = jax.lax.broadcasted_iota(jnp.int32, (N, N), 0)
        jj = jax.lax.broadcasted_iota(jnp.int32, (N, N), 1)
        eye = (ii == jj).astype(jnp.float32)
        pad = jnp.float32(N * N) - jnp.sum(mask)
        delta00 = ((ii == 0) & (jj == 0)).astype(jnp.float32)
        cnt = mask + eye + pad * delta00
        deg = jnp.sum(cnt, axis=0, keepdims=True)
        dinv = jax.lax.rsqrt(deg)
        s = cnt.T * dinv * dinv.reshape(N, 1)
        s_ref[...] = s
        sx_ref[...] = jnp.dot(s, xr_ref[...],
                              preferred_element_type=jnp.float32).astype(jnp.bfloat16)

    @pl.when(j < NTILES)
    def _layer0():
        dsj = pl.ds(j * TILE, TILE)
        t = jnp.dot(sx_ref[...], w0_ref[...].astype(jnp.bfloat16),
                    preferred_element_type=jnp.float32)
        h1_tile = ALPHA * xr_ref[:, dsj] + t + b0_ref[:, dsj]
        h1_ref[:, dsj] = h1_tile
        h1b_ref[:, dsj] = h1_tile.astype(jnp.bfloat16)

    @pl.when(j > 0)
    def _layer1():
        dsp = pl.ds((j - 1) * TILE, TILE)
        part = jnp.dot(h1b_ref[:, dsp], w1_ref[...].astype(jnp.bfloat16),
                       preferred_element_type=jnp.float32)

        @pl.when(j == 1)
        def _init():
            t1_ref[...] = part

        @pl.when(j > 1)
        def _acc():
            t1_ref[...] = t1_ref[...] + part

    @pl.when(j == NTILES)
    def _epilogue():
        h2 = (ALPHA * xr_ref[...]
              + jnp.dot(s_ref[...], t1_ref[...], preferred_element_type=jnp.float32)
              + b1_ref[...])
        wm = wm_ref[...]
        bm = bm_ref[...].reshape(64, 1)
        g0 = xr_ref[...].reshape(BATCH, 64, 512)
        g1 = h1_ref[...].reshape(BATCH, 64, 512)
        g2 = h2.reshape(BATCH, 64, 512)
        for b in range(BATCH):
            ob = (jnp.dot(wm[:, 0:64], g0[b], preferred_element_type=jnp.float32)
                  + jnp.dot(wm[:, 64:128], g1[b], preferred_element_type=jnp.float32)
                  + jnp.dot(wm[:, 128:192], g2[b], preferred_element_type=jnp.float32)
                  + bm)
            out_ref[pl.ds(b * 8, 8), :] = ob.reshape(8, FEAT)


def kernel(X, A, W_g0, b_g0, W_g1, b_g1, W_mlp, b_mlp):
    batch, c, n, seq = X.shape
    Xr = X.reshape(ROWS, FEAT)
    out_r = pl.pallas_call(
        _body,
        grid=(NTILES + 1,),
        in_specs=[
            pl.BlockSpec((ROWS, FEAT), lambda j: (0, 0)),
            pl.BlockSpec((N, N), lambda j: (0, 0)),
            pl.BlockSpec((FEAT, TILE),
                         lambda j: (0, jnp.minimum(j, NTILES - 1))),
            pl.BlockSpec((TILE, FEAT),
                         lambda j: (jnp.maximum(j - 1, 0), 0)),
            pl.BlockSpec((1, FEAT), lambda j: (0, 0)),
            pl.BlockSpec((1, FEAT), lambda j: (0, 0)),
            pl.BlockSpec((64, 192), lambda j: (0, 0)),
            pl.BlockSpec((1, 64), lambda j: (0, 0)),
        ],
        out_specs=pl.BlockSpec((ROWS, FEAT), lambda j: (0, 0)),
        out_shape=jax.ShapeDtypeStruct((ROWS, FEAT), jnp.float32),
        scratch_shapes=[
            pltpu.VMEM((ROWS, FEAT), jnp.float32),
            pltpu.VMEM((ROWS, FEAT), jnp.bfloat16),
            pltpu.VMEM((ROWS, FEAT), jnp.float32),
            pltpu.VMEM((N, N), jnp.float32),
            pltpu.VMEM((ROWS, FEAT), jnp.bfloat16),
        ],
    )(Xr, A, W_g0, W_g1, b_g0.reshape(1, FEAT), b_g1.reshape(1, FEAT),
      W_mlp, b_mlp.reshape(1, 64))
    return out_r.reshape(batch, c, n, seq)
